# baseline (speedup 1.0000x reference)
import jax
import jax.numpy as jnp
from jax.experimental import pallas as pl
from jax.experimental.pallas import tpu as pltpu

_EPS = 1e-6
_MAXR = 4


def _body(sw_ref, x_ref, cen_ref, wid_ref, o_ref):
    xn = x_ref[0]
    H, W, C = xn.shape
    W2, L = W // 2, 2 * C
    half = C
    x4 = xn.reshape(H, W2, 2, C)
    x = jnp.concatenate([x4[:, :, 0, :], x4[:, :, 1, :]], axis=-1)
    xa = jnp.abs(x) + _EPS

    zc = jnp.zeros((H, 2, L), jnp.float32)
    ap = jnp.concatenate([zc, xa, zc], axis=1)

    def s(t):
        return ap[:, 2 + t:2 + t + W2, :]

    def s0(t):
        return ap[:, 2 + t:2 + t + W2, :half]

    def s1(t):
        return ap[:, 2 + t:2 + t + W2, half:]

    def sym(d):
        if d % 2 == 0:
            t = d // 2
            return s(-t) + s(t)
        t = (d - 1) // 2
        return jnp.concatenate([s1(-t - 1) + s1(t), s0(-t) + s0(t + 1)],
                               axis=-1)

    h = xa
    alpha = jnp.zeros((H, W2, L), jnp.float32)
    for r in range(1, _MAXR + 1):
        h = h + sym(r)
        zr = jnp.zeros((r, W2, L), jnp.float32)
        hp = jnp.concatenate([zr, h, zr], axis=0)
        mu = h
        for d in range(1, r + 1):
            mu = mu + hp[r - d:r - d + H] + hp[r + d:r + d + H]
        alpha = alpha + sw_ref[r - 1] * jnp.log(mu)

    K = cen_ref.shape[0]
    acc = jnp.zeros((H, W2, L), jnp.float32)
    for k in range(K):
        ck = cen_ref[k, :].reshape(1, 1, L)
        wk = wid_ref[k, :].reshape(1, 1, L)
        dk = alpha - ck
        acc = acc + jnp.maximum(1.0 - wk * (dk * dk), 0.0)

    res = x + 1.0 / (1.0 + jnp.exp(-acc))
    r4 = jnp.stack([res[:, :, :half], res[:, :, half:]], axis=2)
    o_ref[0] = r4.reshape(H, W, C)


def kernel(x, scale_weights, centers, widths):
    B, H, W, C = x.shape
    K = centers.shape[1]
    L = 2 * C
    cen2 = jnp.tile(centers.T, (1, 2))
    wid2 = jnp.tile(widths.T, (1, 2))

    return pl.pallas_call(
        _body,
        grid=(B,),
        in_specs=[
            pl.BlockSpec(memory_space=pltpu.SMEM),
            pl.BlockSpec((1, H, W, C), lambda b: (b, 0, 0, 0)),
            pl.BlockSpec((K, L), lambda b: (0, 0)),
            pl.BlockSpec((K, L), lambda b: (0, 0)),
        ],
        out_specs=pl.BlockSpec((1, H, W, C), lambda b: (b, 0, 0, 0)),
        out_shape=jax.ShapeDtypeStruct((B, H, W, C), jnp.float32),
        compiler_params=pltpu.CompilerParams(
            dimension_semantics=("parallel",),
            vmem_limit_bytes=52 * 1024 * 1024,
        ),
        name="singularity_hist_recal",
    )(scale_weights, x, cen2, wid2)


# speedup vs baseline: 1.3838x; 1.3838x over previous
import jax
import jax.numpy as jnp
from jax.experimental import pallas as pl
from jax.experimental.pallas import tpu as pltpu

_EPS = 1e-6
_MAXR = 4


def _body(sw_ref, x_ref, cen_ref, wid_ref, o_ref):
    xn = x_ref[0]
    H, W, C = xn.shape
    Wh = W // 2
    L = 2 * C

    x = jnp.concatenate([xn[:, :Wh, :], xn[:, Wh:, :]], axis=-1)
    xa = jnp.abs(x) + _EPS

    zh = jnp.zeros((H, _MAXR, C), jnp.float32)
    el = jnp.concatenate([zh, jnp.abs(xn[:, Wh - _MAXR:Wh, :]) + _EPS],
                         axis=-1)
    er = jnp.concatenate([jnp.abs(xn[:, Wh:Wh + _MAXR, :]) + _EPS, zh],
                         axis=-1)
    ap = jnp.concatenate([el, xa, er], axis=1)

    def s(d):
        return ap[:, _MAXR + d:_MAXR + d + Wh, :]

    h = xa
    alpha = jnp.zeros((H, Wh, L), jnp.float32)
    for r in range(1, _MAXR + 1):
        h = h + s(-r) + s(r)
        zr = jnp.zeros((r, Wh, L), jnp.float32)
        hp = jnp.concatenate([zr, h, zr], axis=0)
        mu = h
        for d in range(1, r + 1):
            mu = mu + hp[r - d:r - d + H] + hp[r + d:r + d + H]
        alpha = alpha + sw_ref[r - 1] * jnp.log(mu)

    K = cen_ref.shape[0]
    acc = jnp.zeros((H, Wh, L), jnp.float32)
    for k in range(K):
        ck = cen_ref[k, :].reshape(1, 1, L)
        wk = wid_ref[k, :].reshape(1, 1, L)
        dk = alpha - ck
        acc = acc + jnp.maximum(1.0 - wk * (dk * dk), 0.0)

    res = x + 1.0 / (1.0 + jnp.exp(-acc))
    o_ref[0] = jnp.concatenate([res[:, :, :C], res[:, :, C:]], axis=1)


def kernel(x, scale_weights, centers, widths):
    B, H, W, C = x.shape
    K = centers.shape[1]
    L = 2 * C
    cen2 = jnp.tile(centers.T, (1, 2))
    wid2 = jnp.tile(widths.T, (1, 2))

    return pl.pallas_call(
        _body,
        grid=(B,),
        in_specs=[
            pl.BlockSpec(memory_space=pltpu.SMEM),
            pl.BlockSpec((1, H, W, C), lambda b: (b, 0, 0, 0)),
            pl.BlockSpec((K, L), lambda b: (0, 0)),
            pl.BlockSpec((K, L), lambda b: (0, 0)),
        ],
        out_specs=pl.BlockSpec((1, H, W, C), lambda b: (b, 0, 0, 0)),
        out_shape=jax.ShapeDtypeStruct((B, H, W, C), jnp.float32),
        compiler_params=pltpu.CompilerParams(
            dimension_semantics=("parallel",),
            vmem_limit_bytes=52 * 1024 * 1024,
        ),
        name="singularity_hist_recal",
    )(scale_weights, x, cen2, wid2)


# speedup vs baseline: 1.7677x; 1.2774x over previous
import jax
import jax.numpy as jnp
from jax.experimental import pallas as pl
from jax.experimental.pallas import tpu as pltpu

_EPS = 1e-6
_MAXR = 4


def _body(sw_ref, x_ref, cen_ref, wid_ref, o_ref):
    x = x_ref[0]
    H, C, W = x.shape
    xa = jnp.abs(x) + _EPS

    def zw(d):
        return jnp.zeros((H, C, d), jnp.float32)

    def sym(a, d):
        return (jnp.concatenate([a[:, :, d:], zw(d)], axis=-1)
                + jnp.concatenate([zw(d), a[:, :, :W - d]], axis=-1))

    h = xa
    alpha = jnp.zeros((H, C, W), jnp.float32)
    for r in range(1, _MAXR + 1):
        h = h + sym(xa, r)
        zr = jnp.zeros((r, C, W), jnp.float32)
        hp = jnp.concatenate([zr, h, zr], axis=0)
        mu = h
        for d in range(1, r + 1):
            mu = mu + hp[r - d:r - d + H] + hp[r + d:r + d + H]
        alpha = alpha + sw_ref[r - 1] * jnp.log(mu)

    K = cen_ref.shape[0]
    acc = jnp.zeros((H, C, W), jnp.float32)
    for k in range(K):
        dk = alpha - cen_ref[k]
        acc = acc + jnp.maximum(1.0 - wid_ref[k] * (dk * dk), 0.0)

    o_ref[0] = x + 1.0 / (1.0 + jnp.exp(-acc))


def kernel(x, scale_weights, centers, widths):
    B, H, W, C = x.shape
    K = centers.shape[1]
    xt = jnp.transpose(x, (0, 1, 3, 2))
    cen_b = jnp.broadcast_to(centers.T[:, :, None], (K, C, W))
    wid_b = jnp.broadcast_to(widths.T[:, :, None], (K, C, W))

    out = pl.pallas_call(
        _body,
        grid=(B,),
        in_specs=[
            pl.BlockSpec(memory_space=pltpu.SMEM),
            pl.BlockSpec((1, H, C, W), lambda b: (b, 0, 0, 0)),
            pl.BlockSpec((K, C, W), lambda b: (0, 0, 0)),
            pl.BlockSpec((K, C, W), lambda b: (0, 0, 0)),
        ],
        out_specs=pl.BlockSpec((1, H, C, W), lambda b: (b, 0, 0, 0)),
        out_shape=jax.ShapeDtypeStruct((B, H, C, W), jnp.float32),
        compiler_params=pltpu.CompilerParams(
            dimension_semantics=("parallel",),
            vmem_limit_bytes=52 * 1024 * 1024,
        ),
        name="singularity_hist_recal",
    )(scale_weights, xt, cen_b, wid_b)
    return jnp.transpose(out, (0, 1, 3, 2))


# speedup vs baseline: 1.8672x; 1.0563x over previous
import jax
import jax.numpy as jnp
from jax.experimental import pallas as pl
from jax.experimental.pallas import tpu as pltpu

_EPS = 1e-6
_MAXR = 4


def _body(sw_ref, x_ref, cen_ref, wid_ref, o_ref):
    x = x_ref[0]
    H, C, W = x.shape
    xa = jnp.abs(x) + _EPS

    zw = jnp.zeros((H, C, _MAXR), jnp.float32)
    ax = jnp.concatenate([zw, xa, zw], axis=-1)

    def s(d):
        return ax[:, :, _MAXR + d:_MAXR + d + W]

    h = xa
    alpha = jnp.zeros((H, C, W), jnp.float32)
    for r in range(1, _MAXR + 1):
        h = h + s(-r) + s(r)
        zr = jnp.zeros((r, C, W), jnp.float32)
        hp = jnp.concatenate([zr, h, zr], axis=0)
        mu = h
        for d in range(1, r + 1):
            mu = mu + hp[r - d:r - d + H] + hp[r + d:r + d + H]
        alpha = alpha + sw_ref[r - 1] * jnp.log2(mu)

    K = cen_ref.shape[0]
    acc = jnp.zeros((H, C, W), jnp.float32)
    for k in range(K):
        dk = alpha - cen_ref[k]
        acc = acc + jnp.minimum(wid_ref[k] * (dk * dk), 1.0)

    o_ref[0] = x + 1.0 / (1.0 + jnp.exp(acc - float(K)))


def kernel(x, scale_weights, centers, widths):
    B, H, W, C = x.shape
    K = centers.shape[1]
    xt = jnp.transpose(x, (0, 1, 3, 2))
    sw2 = scale_weights * jnp.float32(0.6931471805599453)
    cen_b = jnp.broadcast_to(centers.T[:, :, None], (K, C, W))
    wid_b = jnp.broadcast_to(widths.T[:, :, None], (K, C, W))

    out = pl.pallas_call(
        _body,
        grid=(B,),
        in_specs=[
            pl.BlockSpec(memory_space=pltpu.SMEM),
            pl.BlockSpec((1, H, C, W), lambda b: (b, 0, 0, 0)),
            pl.BlockSpec((K, C, W), lambda b: (0, 0, 0)),
            pl.BlockSpec((K, C, W), lambda b: (0, 0, 0)),
        ],
        out_specs=pl.BlockSpec((1, H, C, W), lambda b: (b, 0, 0, 0)),
        out_shape=jax.ShapeDtypeStruct((B, H, C, W), jnp.float32),
        compiler_params=pltpu.CompilerParams(
            dimension_semantics=("parallel",),
            vmem_limit_bytes=52 * 1024 * 1024,
        ),
        name="singularity_hist_recal",
    )(sw2, xt, cen_b, wid_b)
    return jnp.transpose(out, (0, 1, 3, 2))


# speedup vs baseline: 1.8984x; 1.0167x over previous
import jax
import jax.numpy as jnp
from jax.experimental import pallas as pl
from jax.experimental.pallas import tpu as pltpu

_EPS = 1e-6
_MAXR = 4


def _body(sw_ref, x_ref, dcen_ref, wid_ref, o_ref):
    x = x_ref[0]
    H, C, W = x.shape
    xa = jnp.abs(x) + _EPS

    zw = jnp.zeros((H, C, _MAXR), jnp.float32)
    ax = jnp.concatenate([zw, xa, zw], axis=-1)

    def s(d):
        return ax[:, :, _MAXR + d:_MAXR + d + W]

    h = xa
    alpha = jnp.zeros((H, C, W), jnp.float32)
    for r in range(1, _MAXR + 1):
        h = h + s(-r) + s(r)
        zr = jnp.zeros((r, C, W), jnp.float32)
        hp = jnp.concatenate([zr, h, zr], axis=0)
        mu = h
        for d in range(1, r + 1):
            mu = mu + hp[r - d:r - d + H] + hp[r + d:r + d + H]
        alpha = alpha + sw_ref[r - 1] * jnp.log2(mu)

    K = dcen_ref.shape[0]
    dk = alpha - dcen_ref[0]
    acc = jnp.minimum(wid_ref[0] * (dk * dk), 1.0)
    for k in range(1, K):
        dk = dk - dcen_ref[k]
        acc = acc + jnp.minimum(wid_ref[k] * (dk * dk), 1.0)

    o_ref[0] = x + 1.0 / (1.0 + jnp.exp(acc - float(K)))


def kernel(x, scale_weights, centers, widths):
    B, H, W, C = x.shape
    K = centers.shape[1]
    xt = jnp.transpose(x, (0, 1, 3, 2))
    sw2 = scale_weights * jnp.float32(0.6931471805599453)
    cen_t = centers.T
    dcen = jnp.concatenate([cen_t[:1], jnp.diff(cen_t, axis=0)], axis=0)
    dcen_b = jnp.broadcast_to(dcen[:, :, None], (K, C, W))
    wid_b = jnp.broadcast_to(widths.T[:, :, None], (K, C, W))

    out = pl.pallas_call(
        _body,
        grid=(B,),
        in_specs=[
            pl.BlockSpec(memory_space=pltpu.SMEM),
            pl.BlockSpec((1, H, C, W), lambda b: (b, 0, 0, 0)),
            pl.BlockSpec((K, C, W), lambda b: (0, 0, 0)),
            pl.BlockSpec((K, C, W), lambda b: (0, 0, 0)),
        ],
        out_specs=pl.BlockSpec((1, H, C, W), lambda b: (b, 0, 0, 0)),
        out_shape=jax.ShapeDtypeStruct((B, H, C, W), jnp.float32),
        compiler_params=pltpu.CompilerParams(
            dimension_semantics=("parallel",),
            vmem_limit_bytes=52 * 1024 * 1024,
        ),
        name="singularity_hist_recal",
    )(sw2, xt, dcen_b, wid_b)
    return jnp.transpose(out, (0, 1, 3, 2))
